# baseline (speedup 1.0000x reference)
import jax
import jax.numpy as jnp
from jax import lax
from jax.experimental import pallas as pl
from jax.experimental.pallas import tpu as pltpu

_N = 55
_P = 56
_H = 64
_H2 = 128
_L = 5
_PP = _P * _P
_CR = 3.0


def _fwd_kernel(t_ref, d_ref, x_ref, xt_ref,
                scisrj_ref, srj_ref, scisci_ref, selT_ref, agg_ref,
                e2b_ref, e56_ref,
                embW_ref, embb_ref,
                eW1a_ref, eW1b_ref, ewrLR_ref, eweLR_ref,
                eb1_ref, eW2_ref, eb2_ref,
                nW1a_ref, nW1b_ref, nb1_ref, nW2_ref, nb2_ref,
                cW1_ref, cb1_ref, cW2_ref,
                aW_ref, ab_ref,
                out_ref):
    f32 = jnp.float32
    silu = jax.nn.silu

    def mm(a, b):
        return jnp.dot(a, b, preferred_element_type=f32)

    def bf(v):
        return v.astype(jnp.bfloat16)

    jsub = lax.broadcasted_iota(jnp.int32, (_P, _P), 0)
    maskt_cr = (jsub < _N).astype(f32) * _CR

    zc = jnp.zeros((_P - _N, 3), f32)
    zr = jnp.zeros((3, _P - _N), f32)
    x0c = [jnp.concatenate([x_ref[0, s], zc], axis=0) for s in (0, 1)]
    x0r = [jnp.concatenate([xt_ref[0, s], zr], axis=1) for s in (0, 1)]

    h0 = (t_ref[0] * embW_ref[0:1, :] + d_ref[0] * embW_ref[1:2, :]
          + embb_ref[...])
    h = jnp.broadcast_to(
        jnp.concatenate([h0[0:1, :], h0[1:2, :]], axis=1), (_P, _H2))

    def diffs(cc, cr):
        return [cr[k:k + 1, :] - cc[:, k:k + 1] for k in range(3)]

    def radial(e):
        return e[0] * e[0] + e[1] * e[1] + e[2] * e[2]

    e0 = [diffs(x0c[s], x0r[s]) for s in (0, 1)]
    radT0 = [radial(e0[s]) for s in (0, 1)]
    a_ea = bf(mm(srj_ref[...], bf(jnp.concatenate(radT0, axis=1)))
              * scisci_ref[...])

    coord_c = list(x0c)
    coord_r = list(x0r)
    e2, radT = e0, radT0
    for l in range(_L):
        if l:
            e2 = [diffs(coord_c[s], coord_r[s]) for s in (0, 1)]
            radT = [radial(e2[s]) for s in (0, 1)]
        invT = [1.0 / (jnp.sqrt(radT[s]) + 1.0) for s in (0, 1)]

        preI = mm(h, eW1a_ref[l]) + eb1_ref[l]
        preJ = mm(h, eW1b_ref[l])
        a_rad = bf(mm(srj_ref[...], bf(jnp.concatenate(radT, axis=1)))
                   * scisci_ref[...])
        m1 = silu(mm(scisrj_ref[...],
                     bf(jnp.concatenate([preI, preJ], axis=0)))
                  + mm(a_rad, ewrLR_ref[l]) + mm(a_ea, eweLR_ref[l]))
        m2 = silu(mm(bf(m1), eW2_ref[l]) + eb2_ref[l])
        att = jax.nn.sigmoid(mm(bf(m2), aW_ref[l]) + ab_ref[l])
        m = m2 * mm(bf(att), e2b_ref[...])
        m_bf = bf(m)
        cm = silu(mm(m_bf, cW1_ref[l]) + cb1_ref[l])
        cms = mm(bf(cm), cW2_ref[l])
        gates = jnp.tanh(
            mm(selT_ref[...],
               bf(mm(bf(cms), e56_ref[...]) * scisci_ref[...])))
        agg = mm(agg_ref[...], m_bf)
        for s in (0, 1):
            ts = gates[:, s * _P:(s + 1) * _P] * invT[s] * maskt_cr
            drow = jnp.concatenate(
                [jnp.sum(e2[s][k] * ts, axis=0, keepdims=True)
                 for k in range(3)], axis=0)
            coord_r[s] = coord_r[s] + drow
            coord_c[s] = coord_c[s] + drow.T

        hn = silu(mm(h, nW1a_ref[l]) + mm(agg, nW1b_ref[l])
                  + nb1_ref[l])
        h = h + mm(hn, nW2_ref[l]) + nb2_ref[l]

    for s in (0, 1):
        vel = (coord_c[s] - x0c[s])[:_N, :]
        vel = vel - jnp.sum(vel, axis=0, keepdims=True) * (1.0 / _N)
        out_ref[0, s] = vel


def _blockdiag(w):
    z = jnp.zeros_like(w)
    return jnp.concatenate(
        [jnp.concatenate([w, z], axis=2),
         jnp.concatenate([z, w], axis=2)], axis=1)


def _lanes2(b):
    return jnp.concatenate([b, b], axis=2)


def kernel(t, x, d_base, emb_W, emb_b, edge_W1, edge_b1, edge_W2, edge_b2,
           node_W1, node_b1, node_W2, node_b2, coord_W1, coord_b1, coord_W2,
           att_W, att_b):
    B = t.shape[0]
    G = B // 2
    x4 = x.reshape(G, 2, _N, 3)
    xt = jnp.swapaxes(x4, 2, 3)
    t3 = t.reshape(G, 2, 1)
    db3 = d_base.reshape(G, 2, 1)

    pidx = jnp.arange(_PP, dtype=jnp.int32)
    pi, pj = pidx // _P, pidx % _P
    k56 = jnp.arange(_P, dtype=jnp.int32)
    sci = (pi[:, None] == k56[None, :]).astype(jnp.float32)
    srj = (pj[:, None] == k56[None, :]).astype(jnp.float32)
    scisrj = jnp.concatenate([sci, srj], axis=1)
    scisci = jnp.concatenate([sci, sci], axis=1)
    selT = srj.T
    agg_sel = ((pi[None, :] == k56[:, None])
               & (pj[None, :] != k56[:, None])
               & (pj[None, :] < _N)).astype(jnp.float32)
    lane = jnp.arange(_H2)
    e2b = jnp.stack([(lane < _H).astype(jnp.float32),
                     (lane >= _H).astype(jnp.float32)], axis=0)
    lane2 = jnp.arange(2 * _P)
    e56 = jnp.stack([(lane2 < _P).astype(jnp.float32),
                     (lane2 >= _P).astype(jnp.float32)], axis=0)

    def half_stack(w):
        z = jnp.zeros_like(w)
        top = jnp.broadcast_to(
            jnp.concatenate([w, z], axis=2), (_L, _P, _H2))
        bot = jnp.broadcast_to(
            jnp.concatenate([z, w], axis=2), (_L, _P, _H2))
        return jnp.concatenate([top, bot], axis=1)

    ewr = edge_W1[:, 2 * _H:2 * _H + 1, :]
    ewe = edge_W1[:, 2 * _H + 1:, :]
    bf16 = jnp.bfloat16
    operands = (
        t3, db3, x4, xt,
        scisrj.astype(bf16), srj.astype(bf16), scisci,
        selT.astype(bf16), agg_sel.astype(bf16),
        e2b.astype(bf16), e56.astype(bf16),
        emb_W, emb_b.reshape(1, _H),
        _blockdiag(edge_W1[:, :_H, :]), _blockdiag(edge_W1[:, _H:2 * _H, :]),
        half_stack(ewr).astype(bf16), half_stack(ewe).astype(bf16),
        _lanes2(edge_b1[:, None, :]),
        _blockdiag(edge_W2).astype(bf16), _lanes2(edge_b2[:, None, :]),
        _blockdiag(node_W1[:, :_H, :]), _blockdiag(node_W1[:, _H:, :]),
        _lanes2(node_b1[:, None, :]), _blockdiag(node_W2),
        _lanes2(node_b2[:, None, :]),
        _blockdiag(coord_W1).astype(bf16), _lanes2(coord_b1[:, None, :]),
        _blockdiag(coord_W2).astype(bf16),
        _blockdiag(att_W).astype(bf16), att_b[:, None, :],
    )

    def batched(a):
        bs = (1,) + a.shape[1:]
        return pl.BlockSpec(bs, lambda b: (b,) + (0,) * (a.ndim - 1))

    def full(a):
        return pl.BlockSpec(a.shape, lambda b: (0,) * a.ndim)

    in_specs = [batched(o) for o in operands[:4]] + \
               [full(o) for o in operands[4:]]

    out = pl.pallas_call(
        _fwd_kernel,
        grid=(G,),
        in_specs=in_specs,
        out_specs=pl.BlockSpec((1, 2, _N, 3), lambda b: (b, 0, 0, 0)),
        out_shape=jax.ShapeDtypeStruct((G, 2, _N, 3), jnp.float32),
        compiler_params=pltpu.CompilerParams(
            dimension_semantics=("parallel",)),
    )(*operands)
    return out.reshape(B, _N * 3)


# speedup vs baseline: 1.0059x; 1.0059x over previous
import jax
import jax.numpy as jnp
from jax import lax
from jax.experimental import pallas as pl
from jax.experimental.pallas import tpu as pltpu

_N = 55
_P = 56
_H = 64
_H2 = 128
_L = 5
_PP = _P * _P
_CR = 3.0


def _fwd_kernel(t_ref, d_ref, x_ref, xt_ref,
                scisrj_ref, srj_ref, scisci_ref, selT_ref, agg_ref,
                e2b_ref, e56_ref,
                embW_ref, embb_ref,
                eW1a_ref, eW1b_ref, ewrLR_ref, eweLR_ref,
                eb1_ref, eW2_ref, eb2_ref,
                nW1a_ref, nW1b_ref, nb1_ref, nW2_ref, nb2_ref,
                cW1_ref, cb1_ref, cW2_ref,
                aW_ref, ab_ref,
                out_ref):
    f32 = jnp.float32
    silu = jax.nn.silu

    def mm(a, b):
        return jnp.dot(a, b, preferred_element_type=f32)

    jsub = lax.broadcasted_iota(jnp.int32, (_P, _P), 0)
    maskt_cr = (jsub < _N).astype(f32) * _CR

    zc = jnp.zeros((_P - _N, 3), f32)
    zr = jnp.zeros((3, _P - _N), f32)

    def diffs(cc, cr):
        return [cr[k:k + 1, :] - cc[:, k:k + 1] for k in range(3)]

    def radial(e):
        return e[0] * e[0] + e[1] * e[1] + e[2] * e[2]

    h0 = (t_ref[0] * embW_ref[0:1, :] + d_ref[0] * embW_ref[1:2, :]
          + embb_ref[...])
    for g in (0, 1):
        ss = (2 * g, 2 * g + 1)
        x0c = [jnp.concatenate([x_ref[0, s], zc], axis=0) for s in ss]
        x0r = [jnp.concatenate([xt_ref[0, s], zr], axis=1) for s in ss]
        h = jnp.broadcast_to(
            jnp.concatenate([h0[ss[0]:ss[0] + 1, :],
                             h0[ss[1]:ss[1] + 1, :]], axis=1), (_P, _H2))

        e0 = [diffs(x0c[k], x0r[k]) for k in (0, 1)]
        radT0 = [radial(e0[k]) for k in (0, 1)]
        a_ea = mm(srj_ref[...],
                  jnp.concatenate(radT0, axis=1)) * scisci_ref[...]

        coord_c = list(x0c)
        coord_r = list(x0r)
        e2, radT = e0, radT0
        for l in range(_L):
            if l:
                e2 = [diffs(coord_c[k], coord_r[k]) for k in (0, 1)]
                radT = [radial(e2[k]) for k in (0, 1)]
            invT = [1.0 / (jnp.sqrt(radT[k]) + 1.0) for k in (0, 1)]

            preI = mm(h, eW1a_ref[l]) + eb1_ref[l]
            preJ = mm(h, eW1b_ref[l])
            a_rad = mm(srj_ref[...],
                       jnp.concatenate(radT, axis=1)) * scisci_ref[...]
            m1 = silu(mm(scisrj_ref[...],
                         jnp.concatenate([preI, preJ], axis=0))
                      + mm(a_rad, ewrLR_ref[l]) + mm(a_ea, eweLR_ref[l]))
            m2 = silu(mm(m1, eW2_ref[l]) + eb2_ref[l])
            att = jax.nn.sigmoid(mm(m2, aW_ref[l]) + ab_ref[l])
            m = m2 * mm(att, e2b_ref[...])
            cm = silu(mm(m, cW1_ref[l]) + cb1_ref[l])
            cms = mm(cm, cW2_ref[l])
            gates = jnp.tanh(
                mm(selT_ref[...],
                   mm(cms, e56_ref[...]) * scisci_ref[...]))
            agg = mm(agg_ref[...], m)
            for k in (0, 1):
                ts = gates[:, k * _P:(k + 1) * _P] * invT[k] * maskt_cr
                drow = jnp.concatenate(
                    [jnp.sum(e2[k][d] * ts, axis=0, keepdims=True)
                     for d in range(3)], axis=0)
                coord_r[k] = coord_r[k] + drow
                coord_c[k] = coord_c[k] + drow.T

            hn = silu(mm(h, nW1a_ref[l]) + mm(agg, nW1b_ref[l])
                      + nb1_ref[l])
            h = h + mm(hn, nW2_ref[l]) + nb2_ref[l]

        for k in (0, 1):
            vel = (coord_c[k] - x0c[k])[:_N, :]
            vel = vel - jnp.sum(vel, axis=0, keepdims=True) * (1.0 / _N)
            out_ref[0, ss[k]] = vel


def _blockdiag(w):
    z = jnp.zeros_like(w)
    return jnp.concatenate(
        [jnp.concatenate([w, z], axis=2),
         jnp.concatenate([z, w], axis=2)], axis=1)


def _lanes2(b):
    return jnp.concatenate([b, b], axis=2)


def kernel(t, x, d_base, emb_W, emb_b, edge_W1, edge_b1, edge_W2, edge_b2,
           node_W1, node_b1, node_W2, node_b2, coord_W1, coord_b1, coord_W2,
           att_W, att_b):
    B = t.shape[0]
    G = B // 4
    x4 = x.reshape(G, 4, _N, 3)
    xt = jnp.swapaxes(x4, 2, 3)
    t3 = t.reshape(G, 4, 1)
    db3 = d_base.reshape(G, 4, 1)

    pidx = jnp.arange(_PP, dtype=jnp.int32)
    pi, pj = pidx // _P, pidx % _P
    k56 = jnp.arange(_P, dtype=jnp.int32)
    sci = (pi[:, None] == k56[None, :]).astype(jnp.float32)
    srj = (pj[:, None] == k56[None, :]).astype(jnp.float32)
    scisrj = jnp.concatenate([sci, srj], axis=1)
    scisci = jnp.concatenate([sci, sci], axis=1)
    selT = srj.T
    agg_sel = ((pi[None, :] == k56[:, None])
               & (pj[None, :] != k56[:, None])
               & (pj[None, :] < _N)).astype(jnp.float32)
    lane = jnp.arange(_H2)
    e2b = jnp.stack([(lane < _H).astype(jnp.float32),
                     (lane >= _H).astype(jnp.float32)], axis=0)
    lane2 = jnp.arange(2 * _P)
    e56 = jnp.stack([(lane2 < _P).astype(jnp.float32),
                     (lane2 >= _P).astype(jnp.float32)], axis=0)

    def half_stack(w):
        z = jnp.zeros_like(w)
        top = jnp.broadcast_to(
            jnp.concatenate([w, z], axis=2), (_L, _P, _H2))
        bot = jnp.broadcast_to(
            jnp.concatenate([z, w], axis=2), (_L, _P, _H2))
        return jnp.concatenate([top, bot], axis=1)

    ewr = edge_W1[:, 2 * _H:2 * _H + 1, :]
    ewe = edge_W1[:, 2 * _H + 1:, :]
    operands = (
        t3, db3, x4, xt,
        scisrj, srj, scisci, selT, agg_sel, e2b, e56,
        emb_W, emb_b.reshape(1, _H),
        _blockdiag(edge_W1[:, :_H, :]), _blockdiag(edge_W1[:, _H:2 * _H, :]),
        half_stack(ewr), half_stack(ewe),
        _lanes2(edge_b1[:, None, :]),
        _blockdiag(edge_W2), _lanes2(edge_b2[:, None, :]),
        _blockdiag(node_W1[:, :_H, :]), _blockdiag(node_W1[:, _H:, :]),
        _lanes2(node_b1[:, None, :]), _blockdiag(node_W2),
        _lanes2(node_b2[:, None, :]),
        _blockdiag(coord_W1), _lanes2(coord_b1[:, None, :]),
        _blockdiag(coord_W2),
        _blockdiag(att_W), att_b[:, None, :],
    )

    def batched(a):
        bs = (1,) + a.shape[1:]
        return pl.BlockSpec(bs, lambda b: (b,) + (0,) * (a.ndim - 1))

    def full(a):
        return pl.BlockSpec(a.shape, lambda b: (0,) * a.ndim)

    in_specs = [batched(o) for o in operands[:4]] + \
               [full(o) for o in operands[4:]]

    out = pl.pallas_call(
        _fwd_kernel,
        grid=(G,),
        in_specs=in_specs,
        out_specs=pl.BlockSpec((1, 4, _N, 3), lambda b: (b, 0, 0, 0)),
        out_shape=jax.ShapeDtypeStruct((G, 4, _N, 3), jnp.float32),
        compiler_params=pltpu.CompilerParams(
            dimension_semantics=("parallel",)),
    )(*operands)
    return out.reshape(B, _N * 3)


# speedup vs baseline: 1.0130x; 1.0071x over previous
import jax
import jax.numpy as jnp
from jax import lax
from jax.experimental import pallas as pl
from jax.experimental.pallas import tpu as pltpu

_N = 55
_P = 56
_H = 64
_H2 = 128
_L = 5
_PP = _P * _P
_CR = 3.0


def _fwd_kernel(t_ref, d_ref, x_ref, xt_ref,
                scisrj_ref, srj_ref, scisci_ref, selT_ref, agg_ref,
                e2b_ref, e56_ref,
                embW_ref, embb_ref,
                eW1a_ref, eW1b_ref, ewrLR_ref, eweLR_ref,
                eb1_ref, eW2_ref, eb2_ref,
                nW1a_ref, nW1b_ref, nb1_ref, nW2_ref, nb2_ref,
                cW1_ref, cb1_ref, cW2_ref,
                aW_ref, ab_ref,
                out_ref):
    f32 = jnp.float32
    silu = jax.nn.silu

    def mm(a, b):
        return jnp.dot(a, b, preferred_element_type=f32)

    jsub = lax.broadcasted_iota(jnp.int32, (_P, _P), 0)
    maskt_cr = (jsub < _N).astype(f32) * _CR

    zc = jnp.zeros((_P - _N, 3), f32)
    zr = jnp.zeros((3, _P - _N), f32)
    x0c = [jnp.concatenate([x_ref[0, s], zc], axis=0) for s in (0, 1)]
    x0r = [jnp.concatenate([xt_ref[0, s], zr], axis=1) for s in (0, 1)]

    h0 = (t_ref[0] * embW_ref[0:1, :] + d_ref[0] * embW_ref[1:2, :]
          + embb_ref[...])
    h = jnp.broadcast_to(
        jnp.concatenate([h0[0:1, :], h0[1:2, :]], axis=1), (_P, _H2))

    def diffs(cc, cr):
        return [cr[k:k + 1, :] - cc[:, k:k + 1] for k in range(3)]

    def radial(e):
        return e[0] * e[0] + e[1] * e[1] + e[2] * e[2]

    e0 = [diffs(x0c[s], x0r[s]) for s in (0, 1)]
    radT0 = [radial(e0[s]) for s in (0, 1)]
    a_ea = mm(srj_ref[...],
              jnp.concatenate(radT0, axis=1)) * scisci_ref[...]

    coord_c = list(x0c)
    coord_r = list(x0r)
    e2, radT = e0, radT0
    for l in range(_L):
        if l:
            e2 = [diffs(coord_c[s], coord_r[s]) for s in (0, 1)]
            radT = [radial(e2[s]) for s in (0, 1)]
        invT = [1.0 / (jnp.sqrt(radT[s]) + 1.0) for s in (0, 1)]

        preI = mm(h, eW1a_ref[l]) + eb1_ref[l]
        preJ = mm(h, eW1b_ref[l])
        a_rad = mm(srj_ref[...],
                   jnp.concatenate(radT, axis=1)) * scisci_ref[...]
        m1 = silu(mm(scisrj_ref[...],
                     jnp.concatenate([preI, preJ], axis=0))
                  + mm(a_rad, ewrLR_ref[l]) + mm(a_ea, eweLR_ref[l]))
        m2 = silu(mm(m1, eW2_ref[l]) + eb2_ref[l])
        att = jax.nn.sigmoid(mm(m2, aW_ref[l]) + ab_ref[l])
        m = m2 * mm(att, e2b_ref[...])
        cm = silu(mm(m, cW1_ref[l]) + cb1_ref[l])
        cms = mm(cm, cW2_ref[l])
        gates = jnp.tanh(
            mm(selT_ref[...],
               mm(cms, e56_ref[...]) * scisci_ref[...]))
        agg = mm(agg_ref[...], m)
        for s in (0, 1):
            ts = gates[:, s * _P:(s + 1) * _P] * invT[s] * maskt_cr
            drow = jnp.concatenate(
                [jnp.sum(e2[s][k] * ts, axis=0, keepdims=True)
                 for k in range(3)], axis=0)
            coord_r[s] = coord_r[s] + drow
            coord_c[s] = coord_c[s] + drow.T

        hn = silu(mm(h, nW1a_ref[l]) + mm(agg, nW1b_ref[l])
                  + nb1_ref[l])
        h = h + mm(hn, nW2_ref[l]) + nb2_ref[l]

    for s in (0, 1):
        vel = (coord_c[s] - x0c[s])[:_N, :]
        vel = vel - jnp.sum(vel, axis=0, keepdims=True) * (1.0 / _N)
        out_ref[0, s] = vel


def _blockdiag(w):
    z = jnp.zeros_like(w)
    return jnp.concatenate(
        [jnp.concatenate([w, z], axis=2),
         jnp.concatenate([z, w], axis=2)], axis=1)


def _lanes2(b):
    return jnp.concatenate([b, b], axis=2)


def kernel(t, x, d_base, emb_W, emb_b, edge_W1, edge_b1, edge_W2, edge_b2,
           node_W1, node_b1, node_W2, node_b2, coord_W1, coord_b1, coord_W2,
           att_W, att_b):
    B = t.shape[0]
    G = B // 2
    x4 = x.reshape(G, 2, _N, 3)
    xt = jnp.swapaxes(x4, 2, 3)
    t3 = t.reshape(G, 2, 1)
    db3 = d_base.reshape(G, 2, 1)

    pidx = jnp.arange(_PP, dtype=jnp.int32)
    pi, pj = pidx // _P, pidx % _P
    k56 = jnp.arange(_P, dtype=jnp.int32)
    sci = (pi[:, None] == k56[None, :]).astype(jnp.float32)
    srj = (pj[:, None] == k56[None, :]).astype(jnp.float32)
    scisrj = jnp.concatenate([sci, srj], axis=1)
    scisci = jnp.concatenate([sci, sci], axis=1)
    selT = srj.T
    agg_sel = ((pi[None, :] == k56[:, None])
               & (pj[None, :] != k56[:, None])
               & (pj[None, :] < _N)).astype(jnp.float32)
    lane = jnp.arange(_H2)
    e2b = jnp.stack([(lane < _H).astype(jnp.float32),
                     (lane >= _H).astype(jnp.float32)], axis=0)
    lane2 = jnp.arange(2 * _P)
    e56 = jnp.stack([(lane2 < _P).astype(jnp.float32),
                     (lane2 >= _P).astype(jnp.float32)], axis=0)

    def half_stack(w):
        z = jnp.zeros_like(w)
        top = jnp.broadcast_to(
            jnp.concatenate([w, z], axis=2), (_L, _P, _H2))
        bot = jnp.broadcast_to(
            jnp.concatenate([z, w], axis=2), (_L, _P, _H2))
        return jnp.concatenate([top, bot], axis=1)

    ewr = edge_W1[:, 2 * _H:2 * _H + 1, :]
    ewe = edge_W1[:, 2 * _H + 1:, :]
    operands = (
        t3, db3, x4, xt,
        scisrj, srj, scisci, selT, agg_sel, e2b, e56,
        emb_W, emb_b.reshape(1, _H),
        _blockdiag(edge_W1[:, :_H, :]), _blockdiag(edge_W1[:, _H:2 * _H, :]),
        half_stack(ewr), half_stack(ewe),
        _lanes2(edge_b1[:, None, :]),
        _blockdiag(edge_W2), _lanes2(edge_b2[:, None, :]),
        _blockdiag(node_W1[:, :_H, :]), _blockdiag(node_W1[:, _H:, :]),
        _lanes2(node_b1[:, None, :]), _blockdiag(node_W2),
        _lanes2(node_b2[:, None, :]),
        _blockdiag(coord_W1), _lanes2(coord_b1[:, None, :]),
        _blockdiag(coord_W2),
        _blockdiag(att_W), att_b[:, None, :],
    )

    def batched(a):
        bs = (1,) + a.shape[1:]
        return pl.BlockSpec(bs, lambda b: (b,) + (0,) * (a.ndim - 1))

    def full(a):
        return pl.BlockSpec(a.shape, lambda b: (0,) * a.ndim)

    in_specs = [batched(o) for o in operands[:4]] + \
               [full(o) for o in operands[4:]]

    out = pl.pallas_call(
        _fwd_kernel,
        grid=(G,),
        in_specs=in_specs,
        out_specs=pl.BlockSpec((1, 2, _N, 3), lambda b: (b, 0, 0, 0)),
        out_shape=jax.ShapeDtypeStruct((G, 2, _N, 3), jnp.float32),
        compiler_params=pltpu.CompilerParams(
            dimension_semantics=("parallel",)),
    )(*operands)
    return out.reshape(B, _N * 3)
